# baseline (speedup 1.0000x reference)
import jax
import jax.numpy as jnp
from jax.experimental import pallas as pl
from jax.experimental.pallas import tpu as pltpu
from jax.experimental.pallas import tpu_sc as plsc

_B = 64
_H = 512
_C = 1024
_V = 100000
_TV = 3968
_NC = 2
_NJ = 13
_GW = 128
_KS = _H // 2


def _sc_gather(emb_table, idx2d):
    mesh = plsc.ScalarSubcoreMesh(axis_name="c", num_cores=2)
    half = _B // 2

    @pl.kernel(
        out_type=jax.ShapeDtypeStruct((_B, _H), emb_table.dtype),
        mesh=mesh,
        scratch_types=[pltpu.SMEM((1, _B), jnp.int32),
                       pltpu.SemaphoreType.DMA,
                       pltpu.SemaphoreType.DMA],
    )
    def gather_kernel(tbl_hbm, idx_hbm, out_hbm, idx_smem, sem0, sem1):
        core = jax.lax.axis_index("c")
        base = core * half
        pltpu.async_copy(idx_hbm, idx_smem, sem0).wait()

        @pl.loop(0, half)
        def _(i):
            r = base + i
            pltpu.make_async_copy(
                tbl_hbm.at[idx_smem[0, r]], out_hbm.at[r], sem1
            ).start()

        @pl.loop(0, half)
        def _(i):
            r = base + i
            pltpu.make_async_copy(
                tbl_hbm.at[idx_smem[0, r]], out_hbm.at[r], sem1
            ).wait()

    return gather_kernel(emb_table, idx2d)


def _dot_t(a, b):
    return jax.lax.dot_general(
        a, b, (((1,), (1,)), ((), ())), preferred_element_type=jnp.float32
    )


def _fused_body(x_ref, h_ref, ctx_ref, wih_ref, whh_ref, bih_ref, bhh_ref,
                wemb_ref, whid_ref, wctx_ref, bemb_ref, wout_a_ref,
                wout_b_ref, bout_ref, logits_ref, hidden_ref, m_ref):
    H = _H

    @pl.when(pl.program_id(1) == 0)
    def _():
        x = x_ref[...]
        h = h_ref[0]
        gi = _dot_t(x, wih_ref[...]) + bih_ref[...]
        gh = _dot_t(h, whh_ref[...]) + bhh_ref[...]
        r = jax.nn.sigmoid(gi[:, :H] + gh[:, :H])
        z = jax.nn.sigmoid(gi[:, H:2 * H] + gh[:, H:2 * H])
        n = jnp.tanh(gi[:, 2 * H:] + r * gh[:, 2 * H:])
        hn = (1.0 - z) * n + z * h
        hidden_ref[0] = hn
        pre = (_dot_t(x, wemb_ref[...]) + _dot_t(hn, whid_ref[...])
               + _dot_t(ctx_ref[...], wctx_ref[...]) + bemb_ref[...])
        rows = jax.lax.broadcasted_iota(jnp.int32, (2 * H, H), 0)
        cols = jax.lax.broadcasted_iota(jnp.int32, (2 * H, H), 1)
        p_even = (rows == 2 * cols).astype(jnp.bfloat16)
        p_odd = (rows == 2 * cols + 1).astype(jnp.bfloat16)
        pre_bf = pre.astype(jnp.bfloat16)
        me = jax.lax.dot_general(pre_bf, p_even, (((1,), (0,)), ((), ())),
                                 preferred_element_type=jnp.float32)
        mo = jax.lax.dot_general(pre_bf, p_odd, (((1,), (0,)), ((), ())),
                                 preferred_element_type=jnp.float32)
        m_ref[...] = jnp.maximum(me, mo).astype(jnp.bfloat16)

    m = m_ref[...]
    acc = jax.lax.dot_general(
        m[:, :_KS], wout_a_ref[...].astype(jnp.bfloat16),
        (((1,), (1,)), ((), ())), preferred_element_type=jnp.float32,
    )
    acc += jax.lax.dot_general(
        m[:, _KS:], wout_b_ref[...].astype(jnp.bfloat16),
        (((1,), (1,)), ((), ())), preferred_element_type=jnp.float32,
    )
    logits_ref[...] = acc + bout_ref[...]


def _fused_call(x, h3, ctx, w_ih, w_hh, b_ih2, b_hh2, w_emb, w_hid, w_ctx,
                b_emb2, w_out, b_out2):
    full = lambda shape: pl.BlockSpec(shape, lambda c, j: tuple(0 for _ in shape))
    return pl.pallas_call(
        _fused_body,
        grid=(_NC, _NJ),
        in_specs=[
            full((_B, _H)),
            full((1, _B, _H)),
            full((_B, _C)),
            full((3 * _H, _H)),
            full((3 * _H, _H)),
            full((1, 3 * _H)),
            full((1, 3 * _H)),
            full((2 * _H, _H)),
            full((2 * _H, _H)),
            full((2 * _H, _C)),
            full((1, 2 * _H)),
            pl.BlockSpec((_TV, _KS), lambda c, j: (c * _NJ + j, 0)),
            pl.BlockSpec((_TV, _KS), lambda c, j: (c * _NJ + j, 1)),
            pl.BlockSpec((1, _TV), lambda c, j: (0, c * _NJ + j)),
        ],
        out_specs=[
            pl.BlockSpec((_B, _TV), lambda c, j: (0, c * _NJ + j)),
            pl.BlockSpec((1, _B, _H), lambda c, j: (0, 0, 0)),
        ],
        out_shape=[
            jax.ShapeDtypeStruct((_B, _V), jnp.float32),
            jax.ShapeDtypeStruct((1, _B, _H), jnp.float32),
        ],
        scratch_shapes=[pltpu.VMEM((_B, _H), jnp.bfloat16)],
        compiler_params=pltpu.CompilerParams(
            dimension_semantics=("parallel", "arbitrary"),
        ),
    )(x, h3, ctx, w_ih, w_hh, b_ih2, b_hh2, w_emb, w_hid, w_ctx, b_emb2,
      w_out, w_out, b_out2)


def kernel(input_step, last_hidden, context_hidden, emb_table, W_ih, W_hh,
           b_ih, b_hh, W_emb, b_emb, W_hid, W_ctx, W_out, b_out):
    idx2d = input_step.reshape(1, _B).astype(jnp.int32)
    x = _sc_gather(emb_table, idx2d)
    logits, hidden = _fused_call(
        x, last_hidden, context_hidden, W_ih, W_hh,
        b_ih.reshape(1, 3 * _H), b_hh.reshape(1, 3 * _H),
        W_emb, W_hid, W_ctx, b_emb.reshape(1, 2 * _H), W_out,
        b_out.reshape(1, _V),
    )
    return (logits, hidden)


# speedup vs baseline: 1.0058x; 1.0058x over previous
import jax
import jax.numpy as jnp
from jax.experimental import pallas as pl
from jax.experimental.pallas import tpu as pltpu
from jax.experimental.pallas import tpu_sc as plsc

_B = 64
_H = 512
_C = 1024
_V = 100000
_TV = 7680
_NC = 2
_NJ = 7
_GW = 128
_KS = _H // 2


def _sc_gather(emb_table, idx2d):
    mesh = plsc.ScalarSubcoreMesh(axis_name="c", num_cores=2)
    half = _B // 2

    @pl.kernel(
        out_type=jax.ShapeDtypeStruct((_B, _H), emb_table.dtype),
        mesh=mesh,
        scratch_types=[pltpu.SMEM((1, _B), jnp.int32),
                       pltpu.SemaphoreType.DMA,
                       pltpu.SemaphoreType.DMA],
    )
    def gather_kernel(tbl_hbm, idx_hbm, out_hbm, idx_smem, sem0, sem1):
        core = jax.lax.axis_index("c")
        base = core * half
        pltpu.async_copy(idx_hbm, idx_smem, sem0).wait()

        @pl.loop(0, half)
        def _(i):
            r = base + i
            pltpu.make_async_copy(
                tbl_hbm.at[idx_smem[0, r]], out_hbm.at[r], sem1
            ).start()

        @pl.loop(0, half)
        def _(i):
            r = base + i
            pltpu.make_async_copy(
                tbl_hbm.at[idx_smem[0, r]], out_hbm.at[r], sem1
            ).wait()

    return gather_kernel(emb_table, idx2d)


def _dot_t(a, b):
    return jax.lax.dot_general(
        a, b, (((1,), (1,)), ((), ())), preferred_element_type=jnp.float32
    )


def _fused_body(x_ref, h_ref, ctx_ref, wih_ref, whh_ref, bih_ref, bhh_ref,
                wemb_ref, whid_ref, wctx_ref, bemb_ref, wout_a_ref,
                wout_b_ref, bout_ref, logits_ref, hidden_ref, m_ref):
    H = _H

    @pl.when(pl.program_id(1) == 0)
    def _():
        x = x_ref[...]
        h = h_ref[0]
        gi = _dot_t(x, wih_ref[...]) + bih_ref[...]
        gh = _dot_t(h, whh_ref[...]) + bhh_ref[...]
        r = jax.nn.sigmoid(gi[:, :H] + gh[:, :H])
        z = jax.nn.sigmoid(gi[:, H:2 * H] + gh[:, H:2 * H])
        n = jnp.tanh(gi[:, 2 * H:] + r * gh[:, 2 * H:])
        hn = (1.0 - z) * n + z * h
        hidden_ref[0] = hn
        pre = (_dot_t(x, wemb_ref[...]) + _dot_t(hn, whid_ref[...])
               + _dot_t(ctx_ref[...], wctx_ref[...]) + bemb_ref[...])
        rows = jax.lax.broadcasted_iota(jnp.int32, (2 * H, H), 0)
        cols = jax.lax.broadcasted_iota(jnp.int32, (2 * H, H), 1)
        p_even = (rows == 2 * cols).astype(jnp.bfloat16)
        p_odd = (rows == 2 * cols + 1).astype(jnp.bfloat16)
        pre_bf = pre.astype(jnp.bfloat16)
        me = jax.lax.dot_general(pre_bf, p_even, (((1,), (0,)), ((), ())),
                                 preferred_element_type=jnp.float32)
        mo = jax.lax.dot_general(pre_bf, p_odd, (((1,), (0,)), ((), ())),
                                 preferred_element_type=jnp.float32)
        m_ref[...] = jnp.maximum(me, mo).astype(jnp.bfloat16)

    m = m_ref[...]
    acc = jax.lax.dot_general(
        m[:, :_KS], wout_a_ref[...].astype(jnp.bfloat16),
        (((1,), (1,)), ((), ())), preferred_element_type=jnp.float32,
    )
    acc += jax.lax.dot_general(
        m[:, _KS:], wout_b_ref[...].astype(jnp.bfloat16),
        (((1,), (1,)), ((), ())), preferred_element_type=jnp.float32,
    )
    logits_ref[...] = acc + bout_ref[...]


def _fused_call(x, h3, ctx, w_ih, w_hh, b_ih2, b_hh2, w_emb, w_hid, w_ctx,
                b_emb2, w_out, b_out2):
    full = lambda shape: pl.BlockSpec(shape, lambda c, j: tuple(0 for _ in shape))
    return pl.pallas_call(
        _fused_body,
        grid=(_NC, _NJ),
        in_specs=[
            full((_B, _H)),
            full((1, _B, _H)),
            full((_B, _C)),
            full((3 * _H, _H)),
            full((3 * _H, _H)),
            full((1, 3 * _H)),
            full((1, 3 * _H)),
            full((2 * _H, _H)),
            full((2 * _H, _H)),
            full((2 * _H, _C)),
            full((1, 2 * _H)),
            pl.BlockSpec((_TV, _KS), lambda c, j: (c * _NJ + j, 0)),
            pl.BlockSpec((_TV, _KS), lambda c, j: (c * _NJ + j, 1)),
            pl.BlockSpec((1, _TV), lambda c, j: (0, c * _NJ + j)),
        ],
        out_specs=[
            pl.BlockSpec((_B, _TV), lambda c, j: (0, c * _NJ + j)),
            pl.BlockSpec((1, _B, _H), lambda c, j: (0, 0, 0)),
        ],
        out_shape=[
            jax.ShapeDtypeStruct((_B, _V), jnp.float32),
            jax.ShapeDtypeStruct((1, _B, _H), jnp.float32),
        ],
        scratch_shapes=[pltpu.VMEM((_B, _H), jnp.bfloat16)],
        compiler_params=pltpu.CompilerParams(
            dimension_semantics=("parallel", "arbitrary"),
        ),
    )(x, h3, ctx, w_ih, w_hh, b_ih2, b_hh2, w_emb, w_hid, w_ctx, b_emb2,
      w_out, w_out, b_out2)


def kernel(input_step, last_hidden, context_hidden, emb_table, W_ih, W_hh,
           b_ih, b_hh, W_emb, b_emb, W_hid, W_ctx, W_out, b_out):
    idx2d = input_step.reshape(1, _B).astype(jnp.int32)
    x = _sc_gather(emb_table, idx2d)
    logits, hidden = _fused_call(
        x, last_hidden, context_hidden, W_ih, W_hh,
        b_ih.reshape(1, 3 * _H), b_hh.reshape(1, 3 * _H),
        W_emb, W_hid, W_ctx, b_emb.reshape(1, 2 * _H), W_out,
        b_out.reshape(1, _V),
    )
    return (logits, hidden)


# speedup vs baseline: 1.0352x; 1.0292x over previous
import jax
import jax.numpy as jnp
from jax.experimental import pallas as pl
from jax.experimental.pallas import tpu as pltpu
from jax.experimental.pallas import tpu_sc as plsc

_B = 64
_H = 512
_C = 1024
_V = 100000
_TV = 7680
_NC = 2
_NJ = 7
_GW = 128
_KS = _H // 2


def _sc_gather(emb_table, idx2d):
    mesh = plsc.ScalarSubcoreMesh(axis_name="c", num_cores=2)
    half = _B // 2

    @pl.kernel(
        out_type=jax.ShapeDtypeStruct((_B, _H), emb_table.dtype),
        mesh=mesh,
        scratch_types=[pltpu.SMEM((1, _B), jnp.int32),
                       pltpu.SemaphoreType.DMA,
                       pltpu.SemaphoreType.DMA],
    )
    def gather_kernel(tbl_hbm, idx_hbm, out_hbm, idx_smem, sem0, sem1):
        core = jax.lax.axis_index("c")
        base = core * half
        pltpu.async_copy(idx_hbm, idx_smem, sem0).wait()

        @pl.loop(0, half)
        def _(i):
            r = base + i
            pltpu.make_async_copy(
                tbl_hbm.at[idx_smem[0, r]], out_hbm.at[r], sem1
            ).start()

        @pl.loop(0, half)
        def _(i):
            r = base + i
            pltpu.make_async_copy(
                tbl_hbm.at[idx_smem[0, r]], out_hbm.at[r], sem1
            ).wait()

    return gather_kernel(emb_table, idx2d)


def _dot_t(a, b):
    return jax.lax.dot_general(
        a, b, (((1,), (1,)), ((), ())), preferred_element_type=jnp.float32
    )


def _pre_body(h_ref, ctx_ref, whh_ref, bhh_ref, wctx_ref,
              gh_ref, pctx_ref):
    gh_ref[...] = _dot_t(h_ref[0], whh_ref[...]) + bhh_ref[...]
    pctx_ref[...] = _dot_t(ctx_ref[...], wctx_ref[...])


def _pre_call(h3, ctx, w_hh, b_hh2, w_ctx):
    return pl.pallas_call(
        _pre_body,
        in_specs=[
            pl.BlockSpec((1, _B, _H), lambda: (0, 0, 0)),
            pl.BlockSpec((_B, _C), lambda: (0, 0)),
            pl.BlockSpec((3 * _H, _H), lambda: (0, 0)),
            pl.BlockSpec((1, 3 * _H), lambda: (0, 0)),
            pl.BlockSpec((2 * _H, _C), lambda: (0, 0)),
        ],
        out_specs=[
            pl.BlockSpec((_B, 3 * _H), lambda: (0, 0)),
            pl.BlockSpec((_B, 2 * _H), lambda: (0, 0)),
        ],
        out_shape=[
            jax.ShapeDtypeStruct((_B, 3 * _H), jnp.float32),
            jax.ShapeDtypeStruct((_B, 2 * _H), jnp.float32),
        ],
    )(h3, ctx, w_hh, b_hh2, w_ctx)


def _fused_body(x_ref, h_ref, gh_ref, pctx_ref, wih_ref, bih_ref,
                wemb_ref, whid_ref, bemb_ref, wout_a_ref,
                wout_b_ref, bout_ref, logits_ref, hidden_ref, m_ref):
    H = _H

    @pl.when(pl.program_id(1) == 0)
    def _():
        x = x_ref[...]
        h = h_ref[0]
        gi = _dot_t(x, wih_ref[...]) + bih_ref[...]
        gh = gh_ref[...]
        r = jax.nn.sigmoid(gi[:, :H] + gh[:, :H])
        z = jax.nn.sigmoid(gi[:, H:2 * H] + gh[:, H:2 * H])
        n = jnp.tanh(gi[:, 2 * H:] + r * gh[:, 2 * H:])
        hn = (1.0 - z) * n + z * h
        hidden_ref[0] = hn
        pre = (_dot_t(x, wemb_ref[...]) + _dot_t(hn, whid_ref[...])
               + pctx_ref[...] + bemb_ref[...])
        rows = jax.lax.broadcasted_iota(jnp.int32, (2 * H, H), 0)
        cols = jax.lax.broadcasted_iota(jnp.int32, (2 * H, H), 1)
        p_even = (rows == 2 * cols).astype(jnp.bfloat16)
        p_odd = (rows == 2 * cols + 1).astype(jnp.bfloat16)
        pre_bf = pre.astype(jnp.bfloat16)
        me = jax.lax.dot_general(pre_bf, p_even, (((1,), (0,)), ((), ())),
                                 preferred_element_type=jnp.float32)
        mo = jax.lax.dot_general(pre_bf, p_odd, (((1,), (0,)), ((), ())),
                                 preferred_element_type=jnp.float32)
        m_ref[...] = jnp.maximum(me, mo).astype(jnp.bfloat16)

    m = m_ref[...]
    acc = jax.lax.dot_general(
        m[:, :_KS], wout_a_ref[...].astype(jnp.bfloat16),
        (((1,), (1,)), ((), ())), preferred_element_type=jnp.float32,
    )
    acc += jax.lax.dot_general(
        m[:, _KS:], wout_b_ref[...].astype(jnp.bfloat16),
        (((1,), (1,)), ((), ())), preferred_element_type=jnp.float32,
    )
    logits_ref[...] = acc + bout_ref[...]


def _fused_call(x, h3, gh, pctx, w_ih, b_ih2, w_emb, w_hid,
                b_emb2, w_out, b_out2):
    full = lambda shape: pl.BlockSpec(shape, lambda c, j: tuple(0 for _ in shape))
    return pl.pallas_call(
        _fused_body,
        grid=(_NC, _NJ),
        in_specs=[
            full((_B, _H)),
            full((1, _B, _H)),
            full((_B, 3 * _H)),
            full((_B, 2 * _H)),
            full((3 * _H, _H)),
            full((1, 3 * _H)),
            full((2 * _H, _H)),
            full((2 * _H, _H)),
            full((1, 2 * _H)),
            pl.BlockSpec((_TV, _KS), lambda c, j: (c * _NJ + j, 0)),
            pl.BlockSpec((_TV, _KS), lambda c, j: (c * _NJ + j, 1)),
            pl.BlockSpec((1, _TV), lambda c, j: (0, c * _NJ + j)),
        ],
        out_specs=[
            pl.BlockSpec((_B, _TV), lambda c, j: (0, c * _NJ + j)),
            pl.BlockSpec((1, _B, _H), lambda c, j: (0, 0, 0)),
        ],
        out_shape=[
            jax.ShapeDtypeStruct((_B, _V), jnp.float32),
            jax.ShapeDtypeStruct((1, _B, _H), jnp.float32),
        ],
        scratch_shapes=[pltpu.VMEM((_B, _H), jnp.bfloat16)],
        compiler_params=pltpu.CompilerParams(
            dimension_semantics=("parallel", "arbitrary"),
        ),
    )(x, h3, gh, pctx, w_ih, b_ih2, w_emb, w_hid, b_emb2,
      w_out, w_out, b_out2)


def kernel(input_step, last_hidden, context_hidden, emb_table, W_ih, W_hh,
           b_ih, b_hh, W_emb, b_emb, W_hid, W_ctx, W_out, b_out):
    idx2d = input_step.reshape(1, _B).astype(jnp.int32)
    x = _sc_gather(emb_table, idx2d)
    gh, pctx = _pre_call(last_hidden, context_hidden, W_hh,
                         b_hh.reshape(1, 3 * _H), W_ctx)
    logits, hidden = _fused_call(
        x, last_hidden, gh, pctx, W_ih,
        b_ih.reshape(1, 3 * _H), W_emb, W_hid,
        b_emb.reshape(1, 2 * _H), W_out,
        b_out.reshape(1, _V),
    )
    return (logits, hidden)


# speedup vs baseline: 1.0457x; 1.0101x over previous
import jax
import jax.numpy as jnp
from jax.experimental import pallas as pl
from jax.experimental.pallas import tpu as pltpu
from jax.experimental.pallas import tpu_sc as plsc

_B = 64
_H = 512
_C = 1024
_V = 100000
_TV = 7680
_NC = 2
_NJ = 7
_KS = _H // 2


def _sc_gather(emb_table, idx2d):
    mesh = plsc.ScalarSubcoreMesh(axis_name="c", num_cores=2)
    half = _B // 2

    @pl.kernel(
        out_type=jax.ShapeDtypeStruct((_B, _H), emb_table.dtype),
        mesh=mesh,
        scratch_types=[pltpu.SMEM((1, _B), jnp.int32),
                       pltpu.SemaphoreType.DMA,
                       pltpu.SemaphoreType.DMA],
    )
    def gather_kernel(tbl_hbm, idx_hbm, out_hbm, idx_smem, sem0, sem1):
        core = jax.lax.axis_index("c")
        base = core * half
        pltpu.async_copy(idx_hbm, idx_smem, sem0).wait()

        @pl.loop(0, half)
        def _(i):
            r = base + i
            pltpu.make_async_copy(
                tbl_hbm.at[idx_smem[0, r]], out_hbm.at[r], sem1
            ).start()

        @pl.loop(0, half)
        def _(i):
            r = base + i
            pltpu.make_async_copy(
                tbl_hbm.at[idx_smem[0, r]], out_hbm.at[r], sem1
            ).wait()

    return gather_kernel(emb_table, idx2d)


def _dot_t(a, b):
    return jax.lax.dot_general(
        a, b, (((1,), (1,)), ((), ())), preferred_element_type=jnp.float32
    )


def _pre_body(h_ref, ctx_ref, whh_ref, bhh_ref, wctx_ref,
              gh_ref, pctx_ref):
    gh_ref[...] = _dot_t(h_ref[0], whh_ref[...]) + bhh_ref[...]
    pctx_ref[...] = _dot_t(ctx_ref[...], wctx_ref[...])


def _pre_call(h3, ctx, w_hh, b_hh2, w_ctx):
    return pl.pallas_call(
        _pre_body,
        in_specs=[
            pl.BlockSpec((1, _B, _H), lambda: (0, 0, 0)),
            pl.BlockSpec((_B, _C), lambda: (0, 0)),
            pl.BlockSpec((3 * _H, _H), lambda: (0, 0)),
            pl.BlockSpec((1, 3 * _H), lambda: (0, 0)),
            pl.BlockSpec((2 * _H, _C), lambda: (0, 0)),
        ],
        out_specs=[
            pl.BlockSpec((_B, 3 * _H), lambda: (0, 0)),
            pl.BlockSpec((_B, 2 * _H), lambda: (0, 0)),
        ],
        out_shape=[
            jax.ShapeDtypeStruct((_B, 3 * _H), jnp.float32),
            jax.ShapeDtypeStruct((_B, 2 * _H), jnp.float32),
        ],
    )(h3, ctx, w_hh, b_hh2, w_ctx)


def _fused_body(x_ref, h_ref, gh_ref, pctx_ref, wih_ref, bih_ref,
                wemb_ref, whid_ref, bemb_ref, wout_a_ref,
                wout_b_ref, bout_ref, logits_ref, hidden_ref, m_ref):
    H = _H

    @pl.when(pl.program_id(1) == 0)
    def _():
        x = x_ref[...]
        h = h_ref[0]
        gi = _dot_t(x, wih_ref[...]) + bih_ref[...]
        gh = gh_ref[...]
        r = jax.nn.sigmoid(gi[:, :H] + gh[:, :H])
        z = jax.nn.sigmoid(gi[:, H:2 * H] + gh[:, H:2 * H])
        n = jnp.tanh(gi[:, 2 * H:] + r * gh[:, 2 * H:])
        hn = (1.0 - z) * n + z * h
        hidden_ref[0] = hn
        pre = (_dot_t(x, wemb_ref[...]) + _dot_t(hn, whid_ref[...])
               + pctx_ref[...] + bemb_ref[...])
        rows = jax.lax.broadcasted_iota(jnp.int32, (2 * H, H), 0)
        cols = jax.lax.broadcasted_iota(jnp.int32, (2 * H, H), 1)
        p_even = (rows == 2 * cols).astype(jnp.bfloat16)
        p_odd = (rows == 2 * cols + 1).astype(jnp.bfloat16)
        pre_bf = pre.astype(jnp.bfloat16)
        me = jax.lax.dot_general(pre_bf, p_even, (((1,), (0,)), ((), ())),
                                 preferred_element_type=jnp.float32)
        mo = jax.lax.dot_general(pre_bf, p_odd, (((1,), (0,)), ((), ())),
                                 preferred_element_type=jnp.float32)
        m_ref[...] = jnp.maximum(me, mo).astype(jnp.bfloat16)

    m = m_ref[...]
    acc = jax.lax.dot_general(
        m[:, :_KS], wout_a_ref[...].astype(jnp.bfloat16),
        (((1,), (1,)), ((), ())), preferred_element_type=jnp.float32,
    )
    acc += jax.lax.dot_general(
        m[:, _KS:], wout_b_ref[...].astype(jnp.bfloat16),
        (((1,), (1,)), ((), ())), preferred_element_type=jnp.float32,
    )
    logits_ref[...] = acc + bout_ref[...]


def _fused_call(x, h3, gh, pctx, w_ih, b_ih2, w_emb, w_hid,
                b_emb2, w_out, b_out2):
    full = lambda shape: pl.BlockSpec(shape, lambda c, j: tuple(0 for _ in shape))
    return pl.pallas_call(
        _fused_body,
        grid=(_NC, _NJ),
        in_specs=[
            full((_B, _H)),
            full((1, _B, _H)),
            full((_B, 3 * _H)),
            full((_B, 2 * _H)),
            full((3 * _H, _H)),
            full((1, 3 * _H)),
            full((2 * _H, _H)),
            full((2 * _H, _H)),
            full((1, 2 * _H)),
            pl.BlockSpec((_TV, _KS), lambda c, j: (c * _NJ + j, 0)),
            pl.BlockSpec((_TV, _KS), lambda c, j: (c * _NJ + j, 1)),
            pl.BlockSpec((1, _TV), lambda c, j: (0, c * _NJ + j)),
        ],
        out_specs=[
            pl.BlockSpec((_B, _TV), lambda c, j: (0, c * _NJ + j)),
            pl.BlockSpec((1, _B, _H), lambda c, j: (0, 0, 0)),
        ],
        out_shape=[
            jax.ShapeDtypeStruct((_B, _V), jnp.float32),
            jax.ShapeDtypeStruct((1, _B, _H), jnp.float32),
        ],
        scratch_shapes=[pltpu.VMEM((_B, _H), jnp.bfloat16)],
        compiler_params=pltpu.CompilerParams(
            dimension_semantics=("parallel", "arbitrary"),
        ),
    )(x, h3, gh, pctx, w_ih, b_ih2, w_emb, w_hid, b_emb2,
      w_out, w_out, b_out2)


def kernel(input_step, last_hidden, context_hidden, emb_table, W_ih, W_hh,
           b_ih, b_hh, W_emb, b_emb, W_hid, W_ctx, W_out, b_out):
    idx2d = input_step.reshape(1, _B).astype(jnp.int32)
    x = _sc_gather(emb_table, idx2d)
    gh, pctx = _pre_call(last_hidden, context_hidden, W_hh,
                         b_hh.reshape(1, 3 * _H), W_ctx)
    logits, hidden = _fused_call(
        x, last_hidden, gh, pctx, W_ih,
        b_ih.reshape(1, 3 * _H), W_emb, W_hid,
        b_emb.reshape(1, 2 * _H), W_out,
        b_out.reshape(1, _V),
    )
    return (logits, hidden)
